# baseline (speedup 1.0000x reference)
import functools

import jax
import jax.numpy as jnp
from jax import lax
from jax.experimental import pallas as pl
from jax.experimental.pallas import tpu as pltpu
from jax.experimental.pallas import tpu_sc as plsc

ROWS, COLS = 128, 8192
LANES = 16
NUM_CORES, NUM_SUBCORES = 2, 16
NUM_WORKERS = NUM_CORES * NUM_SUBCORES
ROWS_PER_WORKER = ROWS // NUM_WORKERS
NUM_CHUNKS = COLS // LANES
ACCS = 4


def _merge(a, b):
    m_a, p_a = a
    m_b, p_b = b
    take_b = (m_b > m_a) | ((m_b == m_a) & (p_b < p_a))
    return jnp.where(take_b, m_b, m_a), jnp.where(take_b, p_b, p_a)


def _row_argmax(row_ref, r, lane_iota):
    neg_inf = jnp.full((LANES,), -jnp.inf, jnp.float32)
    zero_i = jnp.zeros((LANES,), jnp.int32)
    init = tuple((neg_inf, zero_i) for _ in range(ACCS))

    @plsc.parallel_loop(0, NUM_CHUNKS, step=ACCS, carry=init)
    def accs(i, carry):
        out = []
        for k in range(ACCS):
            m, mi = carry[k]
            v = row_ref[r, pl.ds((i + k) * LANES, LANES)]
            gt = v > m
            jv = jnp.full((LANES,), i + k, jnp.int32)
            out.append((jnp.where(gt, v, m), jnp.where(gt, jv, mi)))
        return tuple(out)

    pairs = [(m, mi * LANES + lane_iota) for (m, mi) in accs]
    while len(pairs) > 1:
        pairs = [_merge(pairs[j], pairs[j + 1])
                 for j in range(0, len(pairs), 2)]
    m, pos = pairs[0]

    gmax = m
    for s in (1, 2, 4, 8):
        shuf = gmax.at[lane_iota ^ s].get(mode="promise_in_bounds")
        gmax = jnp.maximum(gmax, shuf)
    cand = jnp.where(m == gmax, pos, jnp.int32(COLS))
    for s in (1, 2, 4, 8):
        shuf = cand.at[lane_iota ^ s].get(mode="promise_in_bounds")
        cand = jnp.minimum(cand, shuf)
    return cand


def _argmax_onehot_body(in_hbm, out_hbm, ibuf, obuf, isem, osem):
    wid = lax.axis_index("s") * NUM_CORES + lax.axis_index("c")
    base = wid * ROWS_PER_WORKER
    lane_iota = lax.iota(jnp.int32, LANES)
    zeros_v = jnp.zeros((LANES,), jnp.float32)
    ones_v = jnp.ones((LANES,), jnp.float32)
    lane0 = lane_iota == 0

    prefetch = pltpu.async_copy(
        in_hbm.at[pl.ds(base, ROWS_PER_WORKER)], ibuf, isem)

    @plsc.parallel_loop(0, NUM_CHUNKS, step=1, unroll=4)
    def _zinit(j):
        for r in range(ROWS_PER_WORKER):
            obuf[r, pl.ds(j * LANES, LANES)] = zeros_v

    def _do_row(r, carry):
        idx_v = _row_argmax(ibuf, r, lane_iota)
        row_v = jnp.full((LANES,), r, jnp.int32)
        plsc.store_scatter(obuf, [row_v, idx_v], ones_v, mask=lane0)
        pltpu.async_copy(obuf.at[r], out_hbm.at[base + r], osem)
        return carry

    prefetch.wait()
    lax.fori_loop(0, ROWS_PER_WORKER, _do_row, 0)

    for r in range(ROWS_PER_WORKER):
        pltpu.make_async_copy(obuf.at[r], out_hbm.at[base + r], osem).wait()


@jax.jit
def kernel(inputs):
    mesh = plsc.VectorSubcoreMesh(
        core_axis_name="c", subcore_axis_name="s",
        num_cores=NUM_CORES, num_subcores=NUM_SUBCORES,
    )
    run = functools.partial(
        pl.kernel,
        out_type=jax.ShapeDtypeStruct((ROWS, COLS), jnp.float32),
        mesh=mesh,
        scratch_types=[
            pltpu.VMEM((ROWS_PER_WORKER, COLS), jnp.float32),
            pltpu.VMEM((ROWS_PER_WORKER, COLS), jnp.float32),
            pltpu.SemaphoreType.DMA,
            pltpu.SemaphoreType.DMA,
        ],
        compiler_params=pltpu.CompilerParams(needs_layout_passes=False),
    )(_argmax_onehot_body)
    return run(inputs)


# speedup vs baseline: 1.0329x; 1.0329x over previous
import functools

import jax
import jax.numpy as jnp
from jax import lax
from jax.experimental import pallas as pl
from jax.experimental.pallas import tpu as pltpu
from jax.experimental.pallas import tpu_sc as plsc

ROWS, COLS = 128, 8192
LANES = 16
NUM_CORES, NUM_SUBCORES = 2, 16
NUM_WORKERS = NUM_CORES * NUM_SUBCORES
ROWS_PER_WORKER = ROWS // NUM_WORKERS
NUM_CHUNKS = COLS // LANES
ACCS = 16


def _merge(a, b):
    m_a, p_a = a
    m_b, p_b = b
    take_b = (m_b > m_a) | ((m_b == m_a) & (p_b < p_a))
    return jnp.where(take_b, m_b, m_a), jnp.where(take_b, p_b, p_a)


def _row_argmax(row_ref, r, lane_iota):
    neg_inf = jnp.full((LANES,), -jnp.inf, jnp.float32)
    zero_i = jnp.zeros((LANES,), jnp.int32)
    init = tuple((neg_inf, zero_i) for _ in range(ACCS))

    @plsc.parallel_loop(0, NUM_CHUNKS, step=ACCS, carry=init)
    def accs(i, carry):
        out = []
        for k in range(ACCS):
            m, mi = carry[k]
            v = row_ref[r, pl.ds((i + k) * LANES, LANES)]
            gt = v > m
            jv = jnp.full((LANES,), i + k, jnp.int32)
            out.append((jnp.where(gt, v, m), jnp.where(gt, jv, mi)))
        return tuple(out)

    pairs = [(m, mi * LANES + lane_iota) for (m, mi) in accs]
    while len(pairs) > 1:
        pairs = [_merge(pairs[j], pairs[j + 1])
                 for j in range(0, len(pairs), 2)]
    m, pos = pairs[0]

    gmax = m
    for s in (1, 2, 4, 8):
        shuf = gmax.at[lane_iota ^ s].get(mode="promise_in_bounds")
        gmax = jnp.maximum(gmax, shuf)
    cand = jnp.where(m == gmax, pos, jnp.int32(COLS))
    for s in (1, 2, 4, 8):
        shuf = cand.at[lane_iota ^ s].get(mode="promise_in_bounds")
        cand = jnp.minimum(cand, shuf)
    return cand


def _argmax_onehot_body(in_hbm, out_hbm, ibuf, obuf, isem, osem):
    wid = lax.axis_index("s") * NUM_CORES + lax.axis_index("c")
    base = wid * ROWS_PER_WORKER
    lane_iota = lax.iota(jnp.int32, LANES)
    zeros_v = jnp.zeros((LANES,), jnp.float32)
    ones_v = jnp.ones((LANES,), jnp.float32)
    lane0 = lane_iota == 0

    prefetch = pltpu.async_copy(
        in_hbm.at[pl.ds(base, ROWS_PER_WORKER)], ibuf, isem)

    @plsc.parallel_loop(0, NUM_CHUNKS, step=1, unroll=4)
    def _zinit(j):
        for r in range(ROWS_PER_WORKER):
            obuf[r, pl.ds(j * LANES, LANES)] = zeros_v

    def _do_row(r, carry):
        idx_v = _row_argmax(ibuf, r, lane_iota)
        row_v = jnp.full((LANES,), r, jnp.int32)
        plsc.store_scatter(obuf, [row_v, idx_v], ones_v, mask=lane0)
        pltpu.async_copy(obuf.at[r], out_hbm.at[base + r], osem)
        return carry

    prefetch.wait()
    lax.fori_loop(0, ROWS_PER_WORKER, _do_row, 0)

    for r in range(ROWS_PER_WORKER):
        pltpu.make_async_copy(obuf.at[r], out_hbm.at[base + r], osem).wait()


@jax.jit
def kernel(inputs):
    mesh = plsc.VectorSubcoreMesh(
        core_axis_name="c", subcore_axis_name="s",
        num_cores=NUM_CORES, num_subcores=NUM_SUBCORES,
    )
    run = functools.partial(
        pl.kernel,
        out_type=jax.ShapeDtypeStruct((ROWS, COLS), jnp.float32),
        mesh=mesh,
        scratch_types=[
            pltpu.VMEM((ROWS_PER_WORKER, COLS), jnp.float32),
            pltpu.VMEM((ROWS_PER_WORKER, COLS), jnp.float32),
            pltpu.SemaphoreType.DMA,
            pltpu.SemaphoreType.DMA,
        ],
        compiler_params=pltpu.CompilerParams(needs_layout_passes=False),
    )(_argmax_onehot_body)
    return run(inputs)


# speedup vs baseline: 1.0475x; 1.0142x over previous
import functools

import jax
import jax.numpy as jnp
from jax import lax
from jax.experimental import pallas as pl
from jax.experimental.pallas import tpu as pltpu
from jax.experimental.pallas import tpu_sc as plsc

ROWS, COLS = 128, 8192
LANES = 16
NUM_CORES, NUM_SUBCORES = 2, 16
NUM_WORKERS = NUM_CORES * NUM_SUBCORES
ROWS_PER_WORKER = ROWS // NUM_WORKERS
NUM_CHUNKS = COLS // LANES
ACCS = 8


def _merge(a, b):
    m_a, p_a = a
    m_b, p_b = b
    take_b = (m_b > m_a) | ((m_b == m_a) & (p_b < p_a))
    return jnp.where(take_b, m_b, m_a), jnp.where(take_b, p_b, p_a)


def _row_argmax(row_ref, r, lane_iota):
    neg_inf = jnp.full((LANES,), -jnp.inf, jnp.float32)
    zero_i = jnp.zeros((LANES,), jnp.int32)
    init = tuple((neg_inf, zero_i) for _ in range(ACCS))

    @plsc.parallel_loop(0, NUM_CHUNKS, step=ACCS, carry=init)
    def accs(i, carry):
        out = []
        for k in range(ACCS):
            m, mi = carry[k]
            v = row_ref[r, pl.ds((i + k) * LANES, LANES)]
            gt = v > m
            jv = jnp.full((LANES,), i + k, jnp.int32)
            out.append((jnp.where(gt, v, m), jnp.where(gt, jv, mi)))
        return tuple(out)

    pairs = [(m, mi * LANES + lane_iota) for (m, mi) in accs]
    while len(pairs) > 1:
        pairs = [_merge(pairs[j], pairs[j + 1])
                 for j in range(0, len(pairs), 2)]
    m, pos = pairs[0]

    gmax = m
    for s in (1, 2, 4, 8):
        shuf = gmax.at[lane_iota ^ s].get(mode="promise_in_bounds")
        gmax = jnp.maximum(gmax, shuf)
    cand = jnp.where(m == gmax, pos, jnp.int32(COLS))
    for s in (1, 2, 4, 8):
        shuf = cand.at[lane_iota ^ s].get(mode="promise_in_bounds")
        cand = jnp.minimum(cand, shuf)
    return cand


def _argmax_onehot_body(in_hbm, out_hbm, ibuf, obuf, isem, osem):
    wid = lax.axis_index("s") * NUM_CORES + lax.axis_index("c")
    base = wid * ROWS_PER_WORKER
    lane_iota = lax.iota(jnp.int32, LANES)
    zeros_v = jnp.zeros((LANES,), jnp.float32)
    ones_v = jnp.ones((LANES,), jnp.float32)
    lane0 = lane_iota == 0

    prefetch = pltpu.async_copy(
        in_hbm.at[pl.ds(base, ROWS_PER_WORKER)], ibuf, isem)

    @plsc.parallel_loop(0, NUM_CHUNKS, step=1, unroll=4)
    def _zinit(j):
        for r in range(ROWS_PER_WORKER):
            obuf[r, pl.ds(j * LANES, LANES)] = zeros_v

    def _do_row(r, carry):
        idx_v = _row_argmax(ibuf, r, lane_iota)
        row_v = jnp.full((LANES,), r, jnp.int32)
        plsc.store_scatter(obuf, [row_v, idx_v], ones_v, mask=lane0)
        pltpu.async_copy(obuf.at[r], out_hbm.at[base + r], osem)
        return carry

    prefetch.wait()
    lax.fori_loop(0, ROWS_PER_WORKER, _do_row, 0)

    for r in range(ROWS_PER_WORKER):
        pltpu.make_async_copy(obuf.at[r], out_hbm.at[base + r], osem).wait()


@jax.jit
def kernel(inputs):
    mesh = plsc.VectorSubcoreMesh(
        core_axis_name="c", subcore_axis_name="s",
        num_cores=NUM_CORES, num_subcores=NUM_SUBCORES,
    )
    run = functools.partial(
        pl.kernel,
        out_type=jax.ShapeDtypeStruct((ROWS, COLS), jnp.float32),
        mesh=mesh,
        scratch_types=[
            pltpu.VMEM((ROWS_PER_WORKER, COLS), jnp.float32),
            pltpu.VMEM((ROWS_PER_WORKER, COLS), jnp.float32),
            pltpu.SemaphoreType.DMA,
            pltpu.SemaphoreType.DMA,
        ],
        compiler_params=pltpu.CompilerParams(needs_layout_passes=False),
    )(_argmax_onehot_body)
    return run(inputs)
